# baseline (speedup 1.0000x reference)
import functools

import jax
import jax.numpy as jnp
from jax import lax
from jax.experimental import pallas as pl
from jax.experimental.pallas import tpu as pltpu
from jax.experimental.pallas import tpu_sc as plsc

N = 10000
E = 160000
FC = 128
EPAD = 163840
EBLK = 32
EROWS = EPAD // EBLK
NT = 10240
NC, NS = 2, 16
RPW = EROWS // NS
ROWS_W = NT // NS
IB = 8
NBUF = 8


def _make_sc_agg(nchunks, with_deg):
  mesh = plsc.VectorSubcoreMesh(core_axis_name="c", subcore_axis_name="s",
                                num_cores=NC)
  out_type = [jax.ShapeDtypeStruct((NT, FC), jnp.float32) for _ in range(nchunks)]
  if with_deg:
    out_type.append(jax.ShapeDtypeStruct((NT, FC), jnp.float32))
  scratch = [
      pltpu.VMEM((IB, EBLK), jnp.int32),
      pltpu.VMEM((IB, EBLK), jnp.int32),
  ]
  scratch += [pltpu.VMEM((EBLK, FC), jnp.float32) for _ in range(NBUF)]
  scratch.append(pltpu.VMEM_SHARED((NT, FC), jnp.float32))
  scratch += [pltpu.SemaphoreType.DMA for _ in range(NBUF)]

  per_core = max(nchunks // NC, 1)

  @functools.partial(pl.kernel, mesh=mesh, out_type=out_type,
                     scratch_types=scratch)
  def k(*refs):
    tabs = refs[:nchunks]
    src_hbm = refs[nchunks]
    dst_hbm = refs[nchunks + 1]
    z_hbm = refs[nchunks + 2]
    pos = nchunks + 3
    if with_deg:
      ones_hbm = refs[pos]
      pos += 1
    outs = refs[pos:pos + nchunks]
    pos += nchunks
    if with_deg:
      deg_out = refs[pos]
      pos += 1
    src_v = refs[pos]
    dst_v = refs[pos + 1]
    bufs = refs[pos + 2:pos + 2 + NBUF]
    agg_sh = refs[pos + 2 + NBUF]
    sems = refs[pos + 3 + NBUF:pos + 3 + 2 * NBUF]

    cid = lax.axis_index("c")
    sid = lax.axis_index("s")

    for ci in range(nchunks):
      @pl.when(cid == ci // per_core)
      def _(ci=ci):
        tab = tabs[ci]
        pltpu.sync_copy(z_hbm.at[pl.ds(sid * ROWS_W, ROWS_W)],
                        agg_sh.at[pl.ds(sid * ROWS_W, ROWS_W)])
        plsc.subcore_barrier()

        @pl.loop(0, RPW // IB)
        def _(bi):
          pltpu.sync_copy(src_hbm.at[pl.ds(sid * RPW + bi * IB, IB)], src_v)
          pltpu.sync_copy(dst_hbm.at[pl.ds(sid * RPW + bi * IB, IB)], dst_v)

          hs = [None] * NBUF
          for i in range(NBUF - 1):
            hs[i] = pltpu.async_copy(tab.at[src_v.at[i]], bufs[i], sems[i])
          for i in range(IB):
            p = i % NBUF
            if i + NBUF - 1 < IB:
              q = (i + NBUF - 1) % NBUF
              hs[q] = pltpu.async_copy(tab.at[src_v.at[i + NBUF - 1]],
                                       bufs[q], sems[q])
            hs[p].wait()
            pltpu.sync_copy(bufs[p], agg_sh.at[dst_v.at[i]], add=True)

        plsc.subcore_barrier()
        pltpu.sync_copy(agg_sh.at[pl.ds(sid * ROWS_W, ROWS_W)],
                        outs[ci].at[pl.ds(sid * ROWS_W, ROWS_W)])

    if with_deg:
      @pl.when(cid == 0)
      def _():
        pltpu.sync_copy(ones_hbm, bufs[0])
        pltpu.sync_copy(z_hbm.at[pl.ds(sid * ROWS_W, ROWS_W)],
                        agg_sh.at[pl.ds(sid * ROWS_W, ROWS_W)])
      plsc.subcore_barrier()

      @pl.when(cid == 0)
      def _():
        @pl.loop(0, RPW // IB)
        def _(bi):
          pltpu.sync_copy(dst_hbm.at[pl.ds(sid * RPW + bi * IB, IB)], dst_v)
          for i in range(IB):
            pltpu.sync_copy(bufs[0], agg_sh.at[dst_v.at[i]], add=True)
      plsc.subcore_barrier()

      @pl.when(cid == 0)
      def _():
        pltpu.sync_copy(agg_sh.at[pl.ds(sid * ROWS_W, ROWS_W)],
                        deg_out.at[pl.ds(sid * ROWS_W, ROWS_W)])

  return k


_sc_cache = {}


def _get_sc_agg(nchunks, with_deg):
  key = (nchunks, with_deg)
  if key not in _sc_cache:
    _sc_cache[key] = _make_sc_agg(nchunks, with_deg)
  return _sc_cache[key]

_R = 2000


def _recip_deg(deg_blk):
  return 1.0 / jnp.maximum(deg_blk[:, 0:1], 1.0)


def _dot(a, b):
  return jnp.dot(a, b, preferred_element_type=jnp.float32)


def _tc0_body(x0, x1, a0, a1, dg, ws, wn, bb, o0, o1, o2, o3):
  r = _recip_deg(dg[...])
  acc = (_dot(x0[...], ws[0:128, :]) + _dot(x1[...], ws[128:256, :])
         + _dot(a0[...] * r, wn[0:128, :]) + _dot(a1[...] * r, wn[128:256, :])
         + bb[...])
  o0[...] = acc[:, 0:128]
  o1[...] = acc[:, 128:256]
  o2[...] = acc[:, 256:384]
  o3[...] = acc[:, 384:512]


def _tc1_body(x0, x1, x2, x3, a0, a1, a2, a3, dg, ws, wn, bb, wn2,
              xo, y0, y1):
  r = _recip_deg(dg[...])
  acc = bb[...]
  for c, (x, a) in enumerate(((x0, a0), (x1, a1), (x2, a2), (x3, a3))):
    acc = acc + _dot(x[...], ws[pl.ds(c * 128, 128), :])
    acc = acc + _dot(a[...] * r, wn[pl.ds(c * 128, 128), :])
  xo[...] = acc
  y = _dot(acc, wn2[...])
  y0[...] = y[:, 0:128]
  y1[...] = y[:, 128:256]


def _tc2_body(x, a0, a1, dg, ws, bb, o):
  r = _recip_deg(dg[...])
  o[...] = (_dot(x[...], ws[...]) + bb[...]
            + jnp.concatenate([a0[...] * r, a1[...] * r], axis=1))


def _chunk_spec():
  return pl.BlockSpec((_R, FC), lambda i: (i, 0))


def _full_spec(shape):
  return pl.BlockSpec(shape, lambda i: (0, 0))


def _deg_spec():
  return pl.BlockSpec((_R, FC), lambda i: (i, 0))


def _tc0(x0, x1, a0, a1, deg, ws, wn, b):
  return pl.pallas_call(
      _tc0_body,
      grid=(N // _R,),
      in_specs=[_chunk_spec()] * 4 + [_deg_spec(), _full_spec((256, 512)),
                _full_spec((256, 512)), _full_spec((1, 512))],
      out_specs=[_chunk_spec()] * 4,
      out_shape=[jax.ShapeDtypeStruct((N, FC), jnp.float32)] * 4,
  )(x0, x1, a0, a1, deg, ws, wn, b)


def _tc1(xs, aggs, deg, ws, wn, b, wn2):
  return pl.pallas_call(
      _tc1_body,
      grid=(N // _R,),
      in_specs=[_chunk_spec()] * 8 + [_deg_spec(), _full_spec((512, 512)),
                _full_spec((512, 512)), _full_spec((1, 512)),
                _full_spec((512, 256))],
      out_specs=[pl.BlockSpec((_R, 512), lambda i: (i, 0)),
                 _chunk_spec(), _chunk_spec()],
      out_shape=[jax.ShapeDtypeStruct((N, 512), jnp.float32),
                 jax.ShapeDtypeStruct((N, FC), jnp.float32),
                 jax.ShapeDtypeStruct((N, FC), jnp.float32)],
  )(*xs, *aggs, deg, ws, wn, b, wn2)


def _tc2(x, a0, a1, deg, ws, b):
  return pl.pallas_call(
      _tc2_body,
      grid=(N // _R,),
      in_specs=[pl.BlockSpec((_R, 512), lambda i: (i, 0)), _chunk_spec(),
                _chunk_spec(), _deg_spec(), _full_spec((512, 256)),
                _full_spec((1, 256))],
      out_specs=pl.BlockSpec((_R, 256), lambda i: (i, 0)),
      out_shape=jax.ShapeDtypeStruct((N, 256), jnp.float32),
  )(x, a0, a1, deg, ws, b)


def kernel(features, edge_index, W_self_0, W_neigh_0, b_0,
           W_self_1, W_neigh_1, b_1, W_self_2, W_neigh_2, b_2):
  src = edge_index[0].astype(jnp.int32)
  dst = edge_index[1].astype(jnp.int32)
  pad = EPAD - E
  pad_src = (jnp.arange(pad, dtype=jnp.int32) * 37) % N
  pad_dst = N + (jnp.arange(pad, dtype=jnp.int32) % (NT - N))
  src2 = jnp.concatenate([src, pad_src]).reshape(EROWS, EBLK)
  dst2 = jnp.concatenate([dst, pad_dst]).reshape(EROWS, EBLK)
  z128 = jnp.zeros((NT, FC), jnp.float32)
  ones128 = jnp.ones((EBLK, FC), jnp.float32)

  x00 = features[:, 0:128]
  x01 = features[:, 128:256]
  a00, a01, deg = _get_sc_agg(2, True)(x00, x01, src2, dst2, z128, ones128)
  x1c = _tc0(x00, x01, a00, a01, deg, W_self_0, W_neigh_0,
             b_0.reshape(1, -1))
  a1c = _get_sc_agg(4, False)(*x1c, src2, dst2, z128)
  x2, y0, y1 = _tc1(x1c, a1c, deg, W_self_1, W_neigh_1,
                    b_1.reshape(1, -1), W_neigh_2)
  ay0, ay1 = _get_sc_agg(2, False)(y0, y1, src2, dst2, z128)
  return _tc2(x2, ay0, ay1, deg, W_self_2, b_2.reshape(1, -1))


# speedup vs baseline: 1.3728x; 1.3728x over previous
import functools

import jax
import jax.numpy as jnp
from jax import lax
from jax.experimental import pallas as pl
from jax.experimental.pallas import tpu as pltpu
from jax.experimental.pallas import tpu_sc as plsc

N = 10000
E = 160000
FC = 128
EPAD = 163840
EBLK = 80
EROWS = EPAD // EBLK
NT = 10240
NC, NS = 2, 16
RPW = EROWS // NS
ROWS_W = NT // NS
IB = 8
NBUF = 4


def _make_sc_agg(nchunks, with_deg):
  mesh = plsc.VectorSubcoreMesh(core_axis_name="c", subcore_axis_name="s",
                                num_cores=NC)
  out_type = [jax.ShapeDtypeStruct((NT, FC), jnp.float32) for _ in range(nchunks)]
  if with_deg:
    out_type.append(jax.ShapeDtypeStruct((NT, FC), jnp.float32))
  scratch = [
      pltpu.VMEM((IB, EBLK), jnp.int32),
      pltpu.VMEM((IB, EBLK), jnp.int32),
  ]
  scratch += [pltpu.VMEM((EBLK, FC), jnp.float32) for _ in range(NBUF)]
  scratch.append(pltpu.VMEM_SHARED((NT, FC), jnp.float32))
  scratch += [pltpu.SemaphoreType.DMA for _ in range(NBUF)]

  per_core = max(nchunks // NC, 1)

  @functools.partial(pl.kernel, mesh=mesh, out_type=out_type,
                     scratch_types=scratch)
  def k(*refs):
    tabs = refs[:nchunks]
    src_hbm = refs[nchunks]
    dst_hbm = refs[nchunks + 1]
    z_hbm = refs[nchunks + 2]
    pos = nchunks + 3
    if with_deg:
      ones_hbm = refs[pos]
      pos += 1
    outs = refs[pos:pos + nchunks]
    pos += nchunks
    if with_deg:
      deg_out = refs[pos]
      pos += 1
    src_v = refs[pos]
    dst_v = refs[pos + 1]
    bufs = refs[pos + 2:pos + 2 + NBUF]
    agg_sh = refs[pos + 2 + NBUF]
    sems = refs[pos + 3 + NBUF:pos + 3 + 2 * NBUF]

    cid = lax.axis_index("c")
    sid = lax.axis_index("s")

    for ci in range(nchunks):
      @pl.when(cid == ci // per_core)
      def _(ci=ci):
        tab = tabs[ci]
        pltpu.sync_copy(z_hbm.at[pl.ds(sid * ROWS_W, ROWS_W)],
                        agg_sh.at[pl.ds(sid * ROWS_W, ROWS_W)])
        plsc.subcore_barrier()

        @pl.loop(0, RPW // IB)
        def _(bi):
          pltpu.sync_copy(src_hbm.at[pl.ds(sid * RPW + bi * IB, IB)], src_v)
          pltpu.sync_copy(dst_hbm.at[pl.ds(sid * RPW + bi * IB, IB)], dst_v)

          hs = [None] * NBUF
          for i in range(NBUF - 1):
            hs[i] = pltpu.async_copy(tab.at[src_v.at[i]], bufs[i], sems[i])
          for i in range(IB):
            p = i % NBUF
            if i + NBUF - 1 < IB:
              q = (i + NBUF - 1) % NBUF
              hs[q] = pltpu.async_copy(tab.at[src_v.at[i + NBUF - 1]],
                                       bufs[q], sems[q])
            hs[p].wait()
            pltpu.sync_copy(bufs[p], agg_sh.at[dst_v.at[i]], add=True)

        plsc.subcore_barrier()
        pltpu.sync_copy(agg_sh.at[pl.ds(sid * ROWS_W, ROWS_W)],
                        outs[ci].at[pl.ds(sid * ROWS_W, ROWS_W)])

    if with_deg:
      @pl.when(cid == 0)
      def _():
        pltpu.sync_copy(ones_hbm, bufs[0])
        pltpu.sync_copy(z_hbm.at[pl.ds(sid * ROWS_W, ROWS_W)],
                        agg_sh.at[pl.ds(sid * ROWS_W, ROWS_W)])
      plsc.subcore_barrier()

      @pl.when(cid == 0)
      def _():
        @pl.loop(0, RPW // IB)
        def _(bi):
          pltpu.sync_copy(dst_hbm.at[pl.ds(sid * RPW + bi * IB, IB)], dst_v)
          for i in range(IB):
            pltpu.sync_copy(bufs[0], agg_sh.at[dst_v.at[i]], add=True)
      plsc.subcore_barrier()

      @pl.when(cid == 0)
      def _():
        pltpu.sync_copy(agg_sh.at[pl.ds(sid * ROWS_W, ROWS_W)],
                        deg_out.at[pl.ds(sid * ROWS_W, ROWS_W)])

  return k


_sc_cache = {}


def _get_sc_agg(nchunks, with_deg):
  key = (nchunks, with_deg)
  if key not in _sc_cache:
    _sc_cache[key] = _make_sc_agg(nchunks, with_deg)
  return _sc_cache[key]

_R = 2000


def _recip_deg(deg_blk):
  return 1.0 / jnp.maximum(deg_blk[:, 0:1], 1.0)


def _dot(a, b):
  return jnp.dot(a, b, preferred_element_type=jnp.float32)


def _tc0_body(x0, x1, a0, a1, dg, ws, wn, bb, o0, o1, o2, o3):
  r = _recip_deg(dg[...])
  acc = (_dot(x0[...], ws[0:128, :]) + _dot(x1[...], ws[128:256, :])
         + _dot(a0[...] * r, wn[0:128, :]) + _dot(a1[...] * r, wn[128:256, :])
         + bb[...])
  o0[...] = acc[:, 0:128]
  o1[...] = acc[:, 128:256]
  o2[...] = acc[:, 256:384]
  o3[...] = acc[:, 384:512]


def _tc1_body(x0, x1, x2, x3, a0, a1, a2, a3, dg, ws, wn, bb, wn2,
              xo, y0, y1):
  r = _recip_deg(dg[...])
  acc = bb[...]
  for c, (x, a) in enumerate(((x0, a0), (x1, a1), (x2, a2), (x3, a3))):
    acc = acc + _dot(x[...], ws[pl.ds(c * 128, 128), :])
    acc = acc + _dot(a[...] * r, wn[pl.ds(c * 128, 128), :])
  xo[...] = acc
  y = _dot(acc, wn2[...])
  y0[...] = y[:, 0:128]
  y1[...] = y[:, 128:256]


def _tc2_body(x, a0, a1, dg, ws, bb, o):
  r = _recip_deg(dg[...])
  o[...] = (_dot(x[...], ws[...]) + bb[...]
            + jnp.concatenate([a0[...] * r, a1[...] * r], axis=1))


def _chunk_spec():
  return pl.BlockSpec((_R, FC), lambda i: (i, 0))


def _full_spec(shape):
  return pl.BlockSpec(shape, lambda i: (0, 0))


def _deg_spec():
  return pl.BlockSpec((_R, FC), lambda i: (i, 0))


def _tc0(x0, x1, a0, a1, deg, ws, wn, b):
  return pl.pallas_call(
      _tc0_body,
      grid=(N // _R,),
      in_specs=[_chunk_spec()] * 4 + [_deg_spec(), _full_spec((256, 512)),
                _full_spec((256, 512)), _full_spec((1, 512))],
      out_specs=[_chunk_spec()] * 4,
      out_shape=[jax.ShapeDtypeStruct((N, FC), jnp.float32)] * 4,
  )(x0, x1, a0, a1, deg, ws, wn, b)


def _tc1(xs, aggs, deg, ws, wn, b, wn2):
  return pl.pallas_call(
      _tc1_body,
      grid=(N // _R,),
      in_specs=[_chunk_spec()] * 8 + [_deg_spec(), _full_spec((512, 512)),
                _full_spec((512, 512)), _full_spec((1, 512)),
                _full_spec((512, 256))],
      out_specs=[pl.BlockSpec((_R, 512), lambda i: (i, 0)),
                 _chunk_spec(), _chunk_spec()],
      out_shape=[jax.ShapeDtypeStruct((N, 512), jnp.float32),
                 jax.ShapeDtypeStruct((N, FC), jnp.float32),
                 jax.ShapeDtypeStruct((N, FC), jnp.float32)],
  )(*xs, *aggs, deg, ws, wn, b, wn2)


def _tc2(x, a0, a1, deg, ws, b):
  return pl.pallas_call(
      _tc2_body,
      grid=(N // _R,),
      in_specs=[pl.BlockSpec((_R, 512), lambda i: (i, 0)), _chunk_spec(),
                _chunk_spec(), _deg_spec(), _full_spec((512, 256)),
                _full_spec((1, 256))],
      out_specs=pl.BlockSpec((_R, 256), lambda i: (i, 0)),
      out_shape=jax.ShapeDtypeStruct((N, 256), jnp.float32),
  )(x, a0, a1, deg, ws, b)


def kernel(features, edge_index, W_self_0, W_neigh_0, b_0,
           W_self_1, W_neigh_1, b_1, W_self_2, W_neigh_2, b_2):
  src = edge_index[0].astype(jnp.int32)
  dst = edge_index[1].astype(jnp.int32)
  pad = EPAD - E
  pad_src = (jnp.arange(pad, dtype=jnp.int32) * 37) % N
  pad_dst = N + (jnp.arange(pad, dtype=jnp.int32) % (NT - N))
  src2 = jnp.concatenate([src, pad_src]).reshape(EROWS, EBLK)
  dst2 = jnp.concatenate([dst, pad_dst]).reshape(EROWS, EBLK)
  z128 = jnp.zeros((NT, FC), jnp.float32)
  ones128 = jnp.ones((EBLK, FC), jnp.float32)

  x00 = features[:, 0:128]
  x01 = features[:, 128:256]
  a00, a01, deg = _get_sc_agg(2, True)(x00, x01, src2, dst2, z128, ones128)
  x1c = _tc0(x00, x01, a00, a01, deg, W_self_0, W_neigh_0,
             b_0.reshape(1, -1))
  a1c = _get_sc_agg(4, False)(*x1c, src2, dst2, z128)
  x2, y0, y1 = _tc1(x1c, a1c, deg, W_self_1, W_neigh_1,
                    b_1.reshape(1, -1), W_neigh_2)
  ay0, ay1 = _get_sc_agg(2, False)(y0, y1, src2, dst2, z128)
  return _tc2(x2, ay0, ay1, deg, W_self_2, b_2.reshape(1, -1))


# speedup vs baseline: 1.4228x; 1.0365x over previous
import functools

import jax
import jax.numpy as jnp
from jax import lax
from jax.experimental import pallas as pl
from jax.experimental.pallas import tpu as pltpu
from jax.experimental.pallas import tpu_sc as plsc

N = 10000
E = 160000
FC = 128
EPAD = 163840
EBLK = 80
EROWS = EPAD // EBLK
NT = 10240
NC, NS = 2, 16
RPW = EROWS // NS
ROWS_W = NT // NS
IB = 8
NBUF = 4


def _make_sc_agg(nchunks, with_deg):
  mesh = plsc.VectorSubcoreMesh(core_axis_name="c", subcore_axis_name="s",
                                num_cores=NC)
  out_type = [jax.ShapeDtypeStruct((NT, FC), jnp.float32) for _ in range(nchunks)]
  if with_deg:
    out_type += [jax.ShapeDtypeStruct((NT, FC), jnp.float32)] * 2
  scratch = [
      pltpu.VMEM((IB, EBLK), jnp.int32),
      pltpu.VMEM((IB, EBLK), jnp.int32),
  ]
  scratch += [pltpu.VMEM((EBLK, FC), jnp.float32) for _ in range(NBUF)]
  scratch.append(pltpu.VMEM_SHARED((NT, FC), jnp.float32))
  scratch += [pltpu.SemaphoreType.DMA for _ in range(NBUF)]

  per_core = max(nchunks // NC, 1)

  @functools.partial(pl.kernel, mesh=mesh, out_type=out_type,
                     scratch_types=scratch)
  def k(*refs):
    tabs = refs[:nchunks]
    src_hbm = refs[nchunks]
    dst_hbm = refs[nchunks + 1]
    z_hbm = refs[nchunks + 2]
    pos = nchunks + 3
    if with_deg:
      ones_hbm = refs[pos]
      pos += 1
    outs = refs[pos:pos + nchunks]
    pos += nchunks
    if with_deg:
      deg_outs = refs[pos:pos + 2]
      pos += 2
    src_v = refs[pos]
    dst_v = refs[pos + 1]
    bufs = refs[pos + 2:pos + 2 + NBUF]
    agg_sh = refs[pos + 2 + NBUF]
    sems = refs[pos + 3 + NBUF:pos + 3 + 2 * NBUF]

    cid = lax.axis_index("c")
    sid = lax.axis_index("s")

    for ci in range(nchunks):
      @pl.when(cid == ci // per_core)
      def _(ci=ci):
        tab = tabs[ci]
        pltpu.sync_copy(z_hbm.at[pl.ds(sid * ROWS_W, ROWS_W)],
                        agg_sh.at[pl.ds(sid * ROWS_W, ROWS_W)])
        plsc.subcore_barrier()

        @pl.loop(0, RPW // IB)
        def _(bi):
          pltpu.sync_copy(src_hbm.at[pl.ds(sid * RPW + bi * IB, IB)], src_v)
          pltpu.sync_copy(dst_hbm.at[pl.ds(sid * RPW + bi * IB, IB)], dst_v)

          hs = [None] * NBUF
          for i in range(NBUF - 1):
            hs[i] = pltpu.async_copy(tab.at[src_v.at[i]], bufs[i], sems[i])
          for i in range(IB):
            p = i % NBUF
            if i + NBUF - 1 < IB:
              q = (i + NBUF - 1) % NBUF
              hs[q] = pltpu.async_copy(tab.at[src_v.at[i + NBUF - 1]],
                                       bufs[q], sems[q])
            hs[p].wait()
            pltpu.sync_copy(bufs[p], agg_sh.at[dst_v.at[i]], add=True)

        plsc.subcore_barrier()
        pltpu.sync_copy(agg_sh.at[pl.ds(sid * ROWS_W, ROWS_W)],
                        outs[ci].at[pl.ds(sid * ROWS_W, ROWS_W)])

    if with_deg:
      pltpu.sync_copy(ones_hbm, bufs[0])
      pltpu.sync_copy(z_hbm.at[pl.ds(sid * ROWS_W, ROWS_W)],
                      agg_sh.at[pl.ds(sid * ROWS_W, ROWS_W)])
      plsc.subcore_barrier()

      half = RPW // 2

      @pl.loop(0, half // IB)
      def _(bi):
        pltpu.sync_copy(
            dst_hbm.at[pl.ds(sid * RPW + cid * half + bi * IB, IB)], dst_v)
        for i in range(IB):
          pltpu.sync_copy(bufs[0], agg_sh.at[dst_v.at[i]], add=True)
      plsc.subcore_barrier()

      for g in range(NC):
        @pl.when(cid == g)
        def _(g=g):
          pltpu.sync_copy(agg_sh.at[pl.ds(sid * ROWS_W, ROWS_W)],
                          deg_outs[g].at[pl.ds(sid * ROWS_W, ROWS_W)])

  return k


_sc_cache = {}


def _get_sc_agg(nchunks, with_deg):
  key = (nchunks, with_deg)
  if key not in _sc_cache:
    _sc_cache[key] = _make_sc_agg(nchunks, with_deg)
  return _sc_cache[key]

_R = 2000


def _recip_deg(d0_blk, d1_blk):
  return 1.0 / jnp.maximum(d0_blk[:, 0:1] + d1_blk[:, 0:1], 1.0)


def _dot(a, b):
  return jnp.dot(a.astype(jnp.bfloat16), b.astype(jnp.bfloat16),
                 preferred_element_type=jnp.float32)


def _tc0_body(x0, x1, a0, a1, d0, d1, ws, wn, bb, o0, o1, o2, o3):
  r = _recip_deg(d0[...], d1[...])
  acc = (_dot(x0[...], ws[0:128, :]) + _dot(x1[...], ws[128:256, :])
         + _dot(a0[...] * r, wn[0:128, :]) + _dot(a1[...] * r, wn[128:256, :])
         + bb[...])
  o0[...] = acc[:, 0:128]
  o1[...] = acc[:, 128:256]
  o2[...] = acc[:, 256:384]
  o3[...] = acc[:, 384:512]


def _tc1_body(x0, x1, x2, x3, a0, a1, a2, a3, d0, d1, ws, wn, bb, wn2,
              xo, y0, y1):
  r = _recip_deg(d0[...], d1[...])
  acc = bb[...]
  for c, (x, a) in enumerate(((x0, a0), (x1, a1), (x2, a2), (x3, a3))):
    acc = acc + _dot(x[...], ws[pl.ds(c * 128, 128), :])
    acc = acc + _dot(a[...] * r, wn[pl.ds(c * 128, 128), :])
  xo[...] = acc
  y = _dot(acc, wn2[...])
  y0[...] = y[:, 0:128]
  y1[...] = y[:, 128:256]


def _tc2_body(x, a0, a1, d0, d1, ws, bb, o):
  r = _recip_deg(d0[...], d1[...])
  o[...] = (_dot(x[...], ws[...]) + bb[...]
            + jnp.concatenate([a0[...] * r, a1[...] * r], axis=1))


def _chunk_spec():
  return pl.BlockSpec((_R, FC), lambda i: (i, 0))


def _full_spec(shape):
  return pl.BlockSpec(shape, lambda i: (0, 0))


def _deg_spec():
  return pl.BlockSpec((_R, FC), lambda i: (i, 0))


def _tc0(x0, x1, a0, a1, d0, d1, ws, wn, b):
  return pl.pallas_call(
      _tc0_body,
      grid=(N // _R,),
      in_specs=[_chunk_spec()] * 4 + [_deg_spec(), _deg_spec(),
                _full_spec((256, 512)), _full_spec((256, 512)),
                _full_spec((1, 512))],
      out_specs=[_chunk_spec()] * 4,
      out_shape=[jax.ShapeDtypeStruct((N, FC), jnp.float32)] * 4,
  )(x0, x1, a0, a1, d0, d1, ws, wn, b)


def _tc1(xs, aggs, d0, d1, ws, wn, b, wn2):
  return pl.pallas_call(
      _tc1_body,
      grid=(N // _R,),
      in_specs=[_chunk_spec()] * 8 + [_deg_spec(), _deg_spec(),
                _full_spec((512, 512)), _full_spec((512, 512)),
                _full_spec((1, 512)), _full_spec((512, 256))],
      out_specs=[pl.BlockSpec((_R, 512), lambda i: (i, 0)),
                 _chunk_spec(), _chunk_spec()],
      out_shape=[jax.ShapeDtypeStruct((N, 512), jnp.float32),
                 jax.ShapeDtypeStruct((N, FC), jnp.float32),
                 jax.ShapeDtypeStruct((N, FC), jnp.float32)],
  )(*xs, *aggs, d0, d1, ws, wn, b, wn2)


def _tc2(x, a0, a1, d0, d1, ws, b):
  return pl.pallas_call(
      _tc2_body,
      grid=(N // _R,),
      in_specs=[pl.BlockSpec((_R, 512), lambda i: (i, 0)), _chunk_spec(),
                _chunk_spec(), _deg_spec(), _deg_spec(),
                _full_spec((512, 256)), _full_spec((1, 256))],
      out_specs=pl.BlockSpec((_R, 256), lambda i: (i, 0)),
      out_shape=jax.ShapeDtypeStruct((N, 256), jnp.float32),
  )(x, a0, a1, d0, d1, ws, b)


def kernel(features, edge_index, W_self_0, W_neigh_0, b_0,
           W_self_1, W_neigh_1, b_1, W_self_2, W_neigh_2, b_2):
  src = edge_index[0].astype(jnp.int32)
  dst = edge_index[1].astype(jnp.int32)
  pad = EPAD - E
  pad_src = (jnp.arange(pad, dtype=jnp.int32) * 37) % N
  pad_dst = N + (jnp.arange(pad, dtype=jnp.int32) % (NT - N))
  src2 = jnp.concatenate([src, pad_src]).reshape(EROWS, EBLK)
  dst2 = jnp.concatenate([dst, pad_dst]).reshape(EROWS, EBLK)
  z128 = jnp.zeros((NT, FC), jnp.float32)
  ones128 = jnp.ones((EBLK, FC), jnp.float32)

  x00 = features[:, 0:128]
  x01 = features[:, 128:256]
  a00, a01, dg0, dg1 = _get_sc_agg(2, True)(x00, x01, src2, dst2, z128,
                                            ones128)
  x1c = _tc0(x00, x01, a00, a01, dg0, dg1, W_self_0, W_neigh_0,
             b_0.reshape(1, -1))
  a1c = _get_sc_agg(4, False)(*x1c, src2, dst2, z128)
  x2, y0, y1 = _tc1(x1c, a1c, dg0, dg1, W_self_1, W_neigh_1,
                    b_1.reshape(1, -1), W_neigh_2)
  ay0, ay1 = _get_sc_agg(2, False)(y0, y1, src2, dst2, z128)
  return _tc2(x2, ay0, ay1, dg0, dg1, W_self_2, b_2.reshape(1, -1))


# speedup vs baseline: 1.6476x; 1.1580x over previous
import functools

import jax
import jax.numpy as jnp
from jax import lax
from jax.experimental import pallas as pl
from jax.experimental.pallas import tpu as pltpu
from jax.experimental.pallas import tpu_sc as plsc

N = 10000
E = 160000
FC = 128
EPAD = 163840
EBLK = 80
EROWS = EPAD // EBLK
NT = 10240
NC, NS = 2, 16
RPW = EROWS // NS
ROWS_W = NT // NS
IB = 16
NBUF = 4


def _make_sc_agg(nchunks, with_deg):
  mesh = plsc.VectorSubcoreMesh(core_axis_name="c", subcore_axis_name="s",
                                num_cores=NC)
  out_type = [jax.ShapeDtypeStruct((NT, FC), jnp.float32) for _ in range(nchunks)]
  if with_deg:
    out_type += [jax.ShapeDtypeStruct((NT, FC), jnp.float32)] * 2
  scratch = [
      pltpu.VMEM((IB, EBLK), jnp.int32),
      pltpu.VMEM((IB, EBLK), jnp.int32),
  ]
  scratch += [pltpu.VMEM((EBLK, FC), jnp.float32) for _ in range(NBUF)]
  scratch.append(pltpu.VMEM_SHARED((NT, FC), jnp.float32))
  scratch += [pltpu.SemaphoreType.DMA for _ in range(NBUF)]

  per_core = max(nchunks // NC, 1)

  @functools.partial(pl.kernel, mesh=mesh, out_type=out_type,
                     scratch_types=scratch)
  def k(*refs):
    tabs = refs[:nchunks]
    src_hbm = refs[nchunks]
    dst_hbm = refs[nchunks + 1]
    z_hbm = refs[nchunks + 2]
    pos = nchunks + 3
    if with_deg:
      ones_hbm = refs[pos]
      pos += 1
    outs = refs[pos:pos + nchunks]
    pos += nchunks
    if with_deg:
      deg_outs = refs[pos:pos + 2]
      pos += 2
    src_v = refs[pos]
    dst_v = refs[pos + 1]
    bufs = refs[pos + 2:pos + 2 + NBUF]
    agg_sh = refs[pos + 2 + NBUF]
    sems = refs[pos + 3 + NBUF:pos + 3 + 2 * NBUF]

    cid = lax.axis_index("c")
    sid = lax.axis_index("s")

    for ci in range(nchunks):
      @pl.when(cid == ci // per_core)
      def _(ci=ci):
        tab = tabs[ci]
        pltpu.sync_copy(z_hbm.at[pl.ds(sid * ROWS_W, ROWS_W)],
                        agg_sh.at[pl.ds(sid * ROWS_W, ROWS_W)])
        plsc.subcore_barrier()

        @pl.loop(0, RPW // IB)
        def _(bi):
          pltpu.sync_copy(src_hbm.at[pl.ds(sid * RPW + bi * IB, IB)], src_v)
          pltpu.sync_copy(dst_hbm.at[pl.ds(sid * RPW + bi * IB, IB)], dst_v)

          hs = [None] * NBUF
          for i in range(NBUF - 1):
            hs[i] = pltpu.async_copy(tab.at[src_v.at[i]], bufs[i], sems[i])
          for i in range(IB):
            p = i % NBUF
            if i + NBUF - 1 < IB:
              q = (i + NBUF - 1) % NBUF
              hs[q] = pltpu.async_copy(tab.at[src_v.at[i + NBUF - 1]],
                                       bufs[q], sems[q])
            hs[p].wait()
            pltpu.sync_copy(bufs[p], agg_sh.at[dst_v.at[i]], add=True)

        plsc.subcore_barrier()
        pltpu.sync_copy(agg_sh.at[pl.ds(sid * ROWS_W, ROWS_W)],
                        outs[ci].at[pl.ds(sid * ROWS_W, ROWS_W)])

    if with_deg:
      pltpu.sync_copy(ones_hbm, bufs[0])
      pltpu.sync_copy(z_hbm.at[pl.ds(sid * ROWS_W, ROWS_W)],
                      agg_sh.at[pl.ds(sid * ROWS_W, ROWS_W)])
      plsc.subcore_barrier()

      half = RPW // 2

      @pl.loop(0, half // IB)
      def _(bi):
        pltpu.sync_copy(
            dst_hbm.at[pl.ds(sid * RPW + cid * half + bi * IB, IB)], dst_v)
        for i in range(IB):
          pltpu.sync_copy(bufs[0], agg_sh.at[dst_v.at[i]], add=True)
      plsc.subcore_barrier()

      for g in range(NC):
        @pl.when(cid == g)
        def _(g=g):
          pltpu.sync_copy(agg_sh.at[pl.ds(sid * ROWS_W, ROWS_W)],
                          deg_outs[g].at[pl.ds(sid * ROWS_W, ROWS_W)])

  return k


_sc_cache = {}


def _get_sc_agg(nchunks, with_deg):
  key = (nchunks, with_deg)
  if key not in _sc_cache:
    _sc_cache[key] = _make_sc_agg(nchunks, with_deg)
  return _sc_cache[key]

_R = 2000


def _recip_deg(d0_blk, d1_blk):
  return 1.0 / jnp.maximum(d0_blk[:, 0:1] + d1_blk[:, 0:1], 1.0)


def _dot(a, b):
  return jnp.dot(a.astype(jnp.bfloat16), b.astype(jnp.bfloat16),
                 preferred_element_type=jnp.float32)


def _tc0_body(x0, x1, a0, a1, d0, d1, ws, wn, bb, o0, o1, o2, o3):
  r = _recip_deg(d0[...], d1[...])
  acc = (_dot(x0[...], ws[0:128, :]) + _dot(x1[...], ws[128:256, :])
         + _dot(a0[...] * r, wn[0:128, :]) + _dot(a1[...] * r, wn[128:256, :])
         + bb[...])
  o0[...] = acc[:, 0:128]
  o1[...] = acc[:, 128:256]
  o2[...] = acc[:, 256:384]
  o3[...] = acc[:, 384:512]


def _tc1_body(x0, x1, x2, x3, a0, a1, a2, a3, d0, d1, ws, wn, bb, wn2,
              xo, y0, y1):
  r = _recip_deg(d0[...], d1[...])
  acc = bb[...]
  for c, (x, a) in enumerate(((x0, a0), (x1, a1), (x2, a2), (x3, a3))):
    acc = acc + _dot(x[...], ws[pl.ds(c * 128, 128), :])
    acc = acc + _dot(a[...] * r, wn[pl.ds(c * 128, 128), :])
  xo[...] = acc
  y = _dot(acc, wn2[...])
  y0[...] = y[:, 0:128]
  y1[...] = y[:, 128:256]


def _tc2_body(x, a0, a1, d0, d1, ws, bb, o):
  r = _recip_deg(d0[...], d1[...])
  o[...] = (_dot(x[...], ws[...]) + bb[...]
            + jnp.concatenate([a0[...] * r, a1[...] * r], axis=1))


def _chunk_spec():
  return pl.BlockSpec((_R, FC), lambda i: (i, 0))


def _full_spec(shape):
  return pl.BlockSpec(shape, lambda i: (0, 0))


def _deg_spec():
  return pl.BlockSpec((_R, FC), lambda i: (i, 0))


def _tc0(x0, x1, a0, a1, d0, d1, ws, wn, b):
  return pl.pallas_call(
      _tc0_body,
      grid=(N // _R,),
      in_specs=[_chunk_spec()] * 4 + [_deg_spec(), _deg_spec(),
                _full_spec((256, 512)), _full_spec((256, 512)),
                _full_spec((1, 512))],
      out_specs=[_chunk_spec()] * 4,
      out_shape=[jax.ShapeDtypeStruct((N, FC), jnp.float32)] * 4,
  )(x0, x1, a0, a1, d0, d1, ws, wn, b)


def _tc1(xs, aggs, d0, d1, ws, wn, b, wn2):
  return pl.pallas_call(
      _tc1_body,
      grid=(N // _R,),
      in_specs=[_chunk_spec()] * 8 + [_deg_spec(), _deg_spec(),
                _full_spec((512, 512)), _full_spec((512, 512)),
                _full_spec((1, 512)), _full_spec((512, 256))],
      out_specs=[pl.BlockSpec((_R, 512), lambda i: (i, 0)),
                 _chunk_spec(), _chunk_spec()],
      out_shape=[jax.ShapeDtypeStruct((N, 512), jnp.float32),
                 jax.ShapeDtypeStruct((N, FC), jnp.float32),
                 jax.ShapeDtypeStruct((N, FC), jnp.float32)],
  )(*xs, *aggs, d0, d1, ws, wn, b, wn2)


def _tc2(x, a0, a1, d0, d1, ws, b):
  return pl.pallas_call(
      _tc2_body,
      grid=(N // _R,),
      in_specs=[pl.BlockSpec((_R, 512), lambda i: (i, 0)), _chunk_spec(),
                _chunk_spec(), _deg_spec(), _deg_spec(),
                _full_spec((512, 256)), _full_spec((1, 256))],
      out_specs=pl.BlockSpec((_R, 256), lambda i: (i, 0)),
      out_shape=jax.ShapeDtypeStruct((N, 256), jnp.float32),
  )(x, a0, a1, d0, d1, ws, b)


def kernel(features, edge_index, W_self_0, W_neigh_0, b_0,
           W_self_1, W_neigh_1, b_1, W_self_2, W_neigh_2, b_2):
  src = edge_index[0].astype(jnp.int32)
  dst = edge_index[1].astype(jnp.int32)
  pad = EPAD - E
  pad_src = (jnp.arange(pad, dtype=jnp.int32) * 37) % N
  pad_dst = N + (jnp.arange(pad, dtype=jnp.int32) % (NT - N))
  src2 = jnp.concatenate([src, pad_src]).reshape(EROWS, EBLK)
  dst2 = jnp.concatenate([dst, pad_dst]).reshape(EROWS, EBLK)
  z128 = jnp.zeros((NT, FC), jnp.float32)
  ones128 = jnp.ones((EBLK, FC), jnp.float32)

  x00 = features[:, 0:128]
  x01 = features[:, 128:256]
  a00, a01, dg0, dg1 = _get_sc_agg(2, True)(x00, x01, src2, dst2, z128,
                                            ones128)
  x1c = _tc0(x00, x01, a00, a01, dg0, dg1, W_self_0, W_neigh_0,
             b_0.reshape(1, -1))
  a1c = _get_sc_agg(4, False)(*x1c, src2, dst2, z128)
  x2, y0, y1 = _tc1(x1c, a1c, dg0, dg1, W_self_1, W_neigh_1,
                    b_1.reshape(1, -1), W_neigh_2)
  ay0, ay1 = _get_sc_agg(2, False)(y0, y1, src2, dst2, z128)
  return _tc2(x2, ay0, ay1, dg0, dg1, W_self_2, b_2.reshape(1, -1))
